# baseline (speedup 1.0000x reference)
import jax
import jax.numpy as jnp
from jax import lax
from jax.experimental import pallas as pl
from jax.experimental.pallas import tpu as pltpu

B, N, C = 8, 8192, 64
G = 512
K = 32


def _fps_body(x_ref, fps_ref, c_ref, dists_ref):
    px = x_ref[0]
    py = x_ref[1]
    pz = x_ref[2]
    lane = lax.broadcasted_iota(jnp.int32, (B, N), 1)
    col = lax.broadcasted_iota(jnp.int32, (B, G), 1)

    fps_ref[...] = jnp.zeros((B, G), jnp.int32)
    dists_ref[...] = jnp.full((B, N), jnp.inf, jnp.float32)
    lx0, ly0, lz0 = px[:, 0:1], py[:, 0:1], pz[:, 0:1]
    c_ref[0] = jnp.where(col == 0, lx0, 0.0)
    c_ref[1] = jnp.where(col == 0, ly0, 0.0)
    c_ref[2] = jnp.where(col == 0, lz0, 0.0)

    def body(i, carry):
        lx, ly, lz = carry
        dx = px - lx
        dy = py - ly
        dz = pz - lz
        d = (dx * dx + dy * dy) + dz * dz
        dists = jnp.minimum(dists_ref[...], d)
        dists_ref[...] = dists
        m = jnp.max(dists, axis=1, keepdims=True)
        cand = jnp.where(dists == m, lane, N)
        nxt = jnp.min(cand, axis=1, keepdims=True)
        fps_ref[...] = jnp.where(col == i, nxt, fps_ref[...])
        eq = lane == nxt
        lx2 = jnp.sum(jnp.where(eq, px, 0.0), axis=1, keepdims=True)
        ly2 = jnp.sum(jnp.where(eq, py, 0.0), axis=1, keepdims=True)
        lz2 = jnp.sum(jnp.where(eq, pz, 0.0), axis=1, keepdims=True)
        c_ref[0] = jnp.where(col == i, lx2, c_ref[0])
        c_ref[1] = jnp.where(col == i, ly2, c_ref[1])
        c_ref[2] = jnp.where(col == i, lz2, c_ref[2])
        return (lx2, ly2, lz2)

    lax.fori_loop(1, G, body, (lx0, ly0, lz0))


def _fps_pallas(xyz_t):
    return pl.pallas_call(
        _fps_body,
        out_shape=[
            jax.ShapeDtypeStruct((B, G), jnp.int32),
            jax.ShapeDtypeStruct((3, B, G), jnp.float32),
        ],
        scratch_shapes=[pltpu.VMEM((B, N), jnp.float32)],
    )(xyz_t)


def _knn(query, key, k):
    d2 = (jnp.sum(query ** 2, axis=-1)[:, :, None]
          + jnp.sum(key ** 2, axis=-1)[:, None, :]
          - 2.0 * jnp.einsum('bqd,bkd->bqk', query, key))
    dist = jnp.sqrt(jnp.clip(d2, 0.0, None))
    neg_d, idx = jax.lax.top_k(-dist, k)
    return -neg_d, idx


def kernel(xyz, features):
    batch_size, num_points, _ = xyz.shape
    xyz_t = jnp.transpose(xyz, (2, 0, 1))
    fps_idx, centers_t = _fps_pallas(xyz_t)
    centers = jnp.transpose(centers_t, (1, 2, 0))

    _, knn_idx = _knn(centers, xyz, K)

    batch_offset = (jnp.arange(batch_size) * num_points).reshape(-1, 1, 1)
    knn_idx_flat = (knn_idx + batch_offset).reshape(-1)

    nbr_xyz = xyz.reshape(-1, 3)[knn_idx_flat]
    nbr_xyz = nbr_xyz.reshape(batch_size, G, K, 3)
    nbr_xyz = nbr_xyz - centers[:, :, None, :]

    nbr_feats = features.reshape(-1, features.shape[-1])[knn_idx_flat]
    nbr_feats = nbr_feats.reshape(batch_size, G, K, features.shape[-1])

    center_feats = jnp.take_along_axis(features, fps_idx[:, :, None], axis=1)
    group_feats = jnp.concatenate(
        [nbr_xyz, nbr_feats, nbr_feats - center_feats[:, :, None, :]], axis=-1)
    return group_feats, centers, knn_idx, fps_idx


# speedup vs baseline: 3.9260x; 3.9260x over previous
import jax
import jax.numpy as jnp
from jax import lax
from jax.experimental import pallas as pl
from jax.experimental.pallas import tpu as pltpu

B, N, C = 8, 8192, 64
G = 512
K = 32


def _fps_body(x_ref, fps_ref, c_ref, dists_ref):
    px = x_ref[0]
    py = x_ref[1]
    pz = x_ref[2]
    lane = lax.broadcasted_iota(jnp.int32, (B, N), 1)
    col = lax.broadcasted_iota(jnp.int32, (B, G), 1)

    fps_ref[...] = jnp.zeros((B, G), jnp.int32)
    dists_ref[...] = jnp.full((B, N), jnp.inf, jnp.float32)
    lx0, ly0, lz0 = px[:, 0:1], py[:, 0:1], pz[:, 0:1]
    c_ref[0] = jnp.where(col == 0, lx0, 0.0)
    c_ref[1] = jnp.where(col == 0, ly0, 0.0)
    c_ref[2] = jnp.where(col == 0, lz0, 0.0)

    def body(i, carry):
        lx, ly, lz = carry
        dx = px - lx
        dy = py - ly
        dz = pz - lz
        d = (dx * dx + dy * dy) + dz * dz
        dists = jnp.minimum(dists_ref[...], d)
        dists_ref[...] = dists
        m = jnp.max(dists, axis=1, keepdims=True)
        cand = jnp.where(dists == m, lane, N)
        nxt = jnp.min(cand, axis=1, keepdims=True)
        fps_ref[...] = jnp.where(col == i, nxt, fps_ref[...])
        eq = lane == nxt
        lx2 = jnp.sum(jnp.where(eq, px, 0.0), axis=1, keepdims=True)
        ly2 = jnp.sum(jnp.where(eq, py, 0.0), axis=1, keepdims=True)
        lz2 = jnp.sum(jnp.where(eq, pz, 0.0), axis=1, keepdims=True)
        c_ref[0] = jnp.where(col == i, lx2, c_ref[0])
        c_ref[1] = jnp.where(col == i, ly2, c_ref[1])
        c_ref[2] = jnp.where(col == i, lz2, c_ref[2])
        return (lx2, ly2, lz2)

    lax.fori_loop(1, G, body, (lx0, ly0, lz0))


def _fps_pallas(xyz_t):
    return pl.pallas_call(
        _fps_body,
        out_shape=[
            jax.ShapeDtypeStruct((B, G), jnp.int32),
            jax.ShapeDtypeStruct((3, B, G), jnp.float32),
        ],
        scratch_shapes=[pltpu.VMEM((B, N), jnp.float32)],
    )(xyz_t)


_NCH = 4
_CH = N // _NCH


def _knn_body(k_ref, kb_ref, q_ref, qb_ref, out_ref, d_ref):
    qx = q_ref[0, 0:1, :]
    qy = q_ref[0, 1:2, :]
    qz = q_ref[0, 2:3, :]
    qq = (qx * qx + qy * qy) + qz * qz

    for c in range(_NCH):
        kc = k_ref[0, pl.ds(c * _CH, _CH), :]
        kx = kc[:, 0:1]
        ky = kc[:, 1:2]
        kz = kc[:, 2:3]
        kk = (kx * kx + ky * ky) + kz * kz
        dot = lax.dot_general(kb_ref[0, pl.ds(c * _CH, _CH), :], qb_ref[0],
                              (((1,), (0,)), ((), ())),
                              preferred_element_type=jnp.float32)
        d2 = (qq + kk) - 2.0 * dot
        d_ref[pl.ds(c * _CH, _CH), :] = jnp.sqrt(jnp.maximum(d2, 0.0))

    rows0 = lax.broadcasted_iota(jnp.int32, (_CH, 1), 0)

    def body(j, _):
        m = jnp.full((1, G), jnp.inf, jnp.float32)
        for c in range(_NCH):
            dv = d_ref[pl.ds(c * _CH, _CH), :]
            m = jnp.minimum(m, jnp.min(dv, axis=0, keepdims=True))
        idx = jnp.full((1, G), N, jnp.int32)
        for c in range(_NCH):
            dv = d_ref[pl.ds(c * _CH, _CH), :]
            cand = jnp.where(dv == m, rows0 + (c * _CH), N)
            idx = jnp.minimum(idx, jnp.min(cand, axis=0, keepdims=True))
        out_ref[0, pl.ds(j, 1), :] = idx
        for c in range(_NCH):
            dv = d_ref[pl.ds(c * _CH, _CH), :]
            hit = (rows0 + (c * _CH)) == idx
            d_ref[pl.ds(c * _CH, _CH), :] = jnp.where(hit, jnp.inf, dv)
        return 0

    lax.fori_loop(0, K, body, 0)


def _knn_pallas(keys4, keys4_bf, q4t, q4t_bf):
    return pl.pallas_call(
        _knn_body,
        grid=(B,),
        in_specs=[
            pl.BlockSpec((1, N, 4), lambda b: (b, 0, 0)),
            pl.BlockSpec((1, N, 4), lambda b: (b, 0, 0)),
            pl.BlockSpec((1, 4, G), lambda b: (b, 0, 0)),
            pl.BlockSpec((1, 4, G), lambda b: (b, 0, 0)),
        ],
        out_specs=pl.BlockSpec((1, K, G), lambda b: (b, 0, 0)),
        out_shape=jax.ShapeDtypeStruct((B, K, G), jnp.int32),
        scratch_shapes=[pltpu.VMEM((N, G), jnp.float32)],
    )(keys4, keys4_bf, q4t, q4t_bf)


def kernel(xyz, features):
    batch_size, num_points, _ = xyz.shape
    xyz_t = jnp.transpose(xyz, (2, 0, 1))
    fps_idx, centers_t = _fps_pallas(xyz_t)
    centers = jnp.transpose(centers_t, (1, 2, 0))

    pad1 = jnp.zeros(xyz.shape[:-1] + (1,), xyz.dtype)
    keys4 = jnp.concatenate([xyz, pad1], axis=-1)
    q4t = jnp.concatenate(
        [centers_t, jnp.zeros((1, B, G), xyz.dtype)], axis=0)
    q4t = jnp.transpose(q4t, (1, 0, 2))
    knnT = _knn_pallas(keys4, keys4.astype(jnp.bfloat16), q4t,
                       q4t.astype(jnp.bfloat16))
    knn_idx = jnp.transpose(knnT, (0, 2, 1))

    batch_offset = (jnp.arange(batch_size) * num_points).reshape(-1, 1, 1)
    knn_idx_flat = (knn_idx + batch_offset).reshape(-1)

    nbr_xyz = xyz.reshape(-1, 3)[knn_idx_flat]
    nbr_xyz = nbr_xyz.reshape(batch_size, G, K, 3)
    nbr_xyz = nbr_xyz - centers[:, :, None, :]

    nbr_feats = features.reshape(-1, features.shape[-1])[knn_idx_flat]
    nbr_feats = nbr_feats.reshape(batch_size, G, K, features.shape[-1])

    center_feats = jnp.take_along_axis(features, fps_idx[:, :, None], axis=1)
    group_feats = jnp.concatenate(
        [nbr_xyz, nbr_feats, nbr_feats - center_feats[:, :, None, :]], axis=-1)
    return group_feats, centers, knn_idx, fps_idx


# speedup vs baseline: 4.8566x; 1.2370x over previous
import functools

import jax
import jax.numpy as jnp
from jax import lax
from jax.experimental import pallas as pl
from jax.experimental.pallas import tpu as pltpu
from jax.experimental.pallas import tpu_sc as plsc

B, N, C = 8, 8192, 64
G = 512
K = 32
OUTC = 3 + C + C


def _fps_body(x_ref, fps_ref, c_ref, dists_ref):
    px = x_ref[0]
    py = x_ref[1]
    pz = x_ref[2]
    lane = lax.broadcasted_iota(jnp.int32, (B, N), 1)
    col = lax.broadcasted_iota(jnp.int32, (B, G), 1)

    fps_ref[...] = jnp.zeros((B, G), jnp.int32)
    dists_ref[...] = jnp.full((B, N), jnp.inf, jnp.float32)
    lx0, ly0, lz0 = px[:, 0:1], py[:, 0:1], pz[:, 0:1]
    c_ref[0] = jnp.where(col == 0, lx0, 0.0)
    c_ref[1] = jnp.where(col == 0, ly0, 0.0)
    c_ref[2] = jnp.where(col == 0, lz0, 0.0)

    def body(i, carry):
        lx, ly, lz = carry
        dx = px - lx
        dy = py - ly
        dz = pz - lz
        d = (dx * dx + dy * dy) + dz * dz
        dists = jnp.minimum(dists_ref[...], d)
        dists_ref[...] = dists
        m = jnp.max(dists, axis=1, keepdims=True)
        cand = jnp.where(dists == m, lane, N)
        nxt = jnp.min(cand, axis=1, keepdims=True)
        fps_ref[...] = jnp.where(col == i, nxt, fps_ref[...])
        eq = lane == nxt
        lx2 = jnp.sum(jnp.where(eq, px, 0.0), axis=1, keepdims=True)
        ly2 = jnp.sum(jnp.where(eq, py, 0.0), axis=1, keepdims=True)
        lz2 = jnp.sum(jnp.where(eq, pz, 0.0), axis=1, keepdims=True)
        c_ref[0] = jnp.where(col == i, lx2, c_ref[0])
        c_ref[1] = jnp.where(col == i, ly2, c_ref[1])
        c_ref[2] = jnp.where(col == i, lz2, c_ref[2])
        return (lx2, ly2, lz2)

    lax.fori_loop(1, G, body, (lx0, ly0, lz0))


def _fps_pallas(xyz_t):
    return pl.pallas_call(
        _fps_body,
        out_shape=[
            jax.ShapeDtypeStruct((B, G), jnp.int32),
            jax.ShapeDtypeStruct((3, B, G), jnp.float32),
        ],
        scratch_shapes=[pltpu.VMEM((B, N), jnp.float32)],
    )(xyz_t)


_NCH = 4
_CH = N // _NCH


def _knn_body(k_ref, kb_ref, q_ref, qb_ref, out_ref, d_ref):
    qx = q_ref[0, 0:1, :]
    qy = q_ref[0, 1:2, :]
    qz = q_ref[0, 2:3, :]
    qq = (qx * qx + qy * qy) + qz * qz

    for c in range(_NCH):
        kc = k_ref[0, pl.ds(c * _CH, _CH), :]
        kx = kc[:, 0:1]
        ky = kc[:, 1:2]
        kz = kc[:, 2:3]
        kk = (kx * kx + ky * ky) + kz * kz
        dot = lax.dot_general(kb_ref[0, pl.ds(c * _CH, _CH), :], qb_ref[0],
                              (((1,), (0,)), ((), ())),
                              preferred_element_type=jnp.float32)
        d2 = (qq + kk) - 2.0 * dot
        d_ref[pl.ds(c * _CH, _CH), :] = jnp.sqrt(jnp.maximum(d2, 0.0))

    rows0 = lax.broadcasted_iota(jnp.int32, (_CH, 1), 0)

    def body(j, _):
        m = jnp.full((1, G), jnp.inf, jnp.float32)
        for c in range(_NCH):
            dv = d_ref[pl.ds(c * _CH, _CH), :]
            m = jnp.minimum(m, jnp.min(dv, axis=0, keepdims=True))
        idx = jnp.full((1, G), N, jnp.int32)
        for c in range(_NCH):
            dv = d_ref[pl.ds(c * _CH, _CH), :]
            cand = jnp.where(dv == m, rows0 + (c * _CH), N)
            idx = jnp.minimum(idx, jnp.min(cand, axis=0, keepdims=True))
        out_ref[0, pl.ds(j, 1), :] = idx
        for c in range(_NCH):
            dv = d_ref[pl.ds(c * _CH, _CH), :]
            hit = (rows0 + (c * _CH)) == idx
            d_ref[pl.ds(c * _CH, _CH), :] = jnp.where(hit, jnp.inf, dv)
        return 0

    lax.fori_loop(0, K, body, 0)


def _knn_pallas(keys4, keys4_bf, q4t, q4t_bf):
    return pl.pallas_call(
        _knn_body,
        grid=(B,),
        in_specs=[
            pl.BlockSpec((1, N, 4), lambda b: (b, 0, 0)),
            pl.BlockSpec((1, N, 4), lambda b: (b, 0, 0)),
            pl.BlockSpec((1, 4, G), lambda b: (b, 0, 0)),
            pl.BlockSpec((1, 4, G), lambda b: (b, 0, 0)),
        ],
        out_specs=pl.BlockSpec((1, K, G), lambda b: (b, 0, 0)),
        out_shape=jax.ShapeDtypeStruct((B, K, G), jnp.int32),
        scratch_shapes=[pltpu.VMEM((N, G), jnp.float32)],
    )(keys4, keys4_bf, q4t, q4t_bf)


_NC, _NS = 2, 16
_NW = _NC * _NS
_GPW = (B * G) // _NW
_GC = 8
_NCHUNK = _GPW // _GC
_ROWS = _GC * K


def _group_body(feats_hbm, xyzp_hbm, knn_hbm, fps_hbm, cent_hbm, out_hbm,
                idx_v, frows, xrows, cfidx, cfeat, cxyz, outbuf, sem):
    wid = lax.axis_index("s") * _NC + lax.axis_index("c")

    def chunk(ch, _):
        g0 = wid * _GPW + ch * _GC
        pltpu.sync_copy(knn_hbm.at[pl.ds(g0 * K, _ROWS)], idx_v)
        pltpu.sync_copy(fps_hbm.at[pl.ds(g0, _GC)], cfidx)
        pltpu.sync_copy(cent_hbm.at[pl.ds(g0, _GC)], cxyz)
        pltpu.async_copy(feats_hbm.at[idx_v], frows, sem).wait()
        pltpu.async_copy(xyzp_hbm.at[idx_v], xrows, sem).wait()
        pltpu.async_copy(feats_hbm.at[cfidx], cfeat, sem).wait()

        for g in range(_GC):
            cpat = cxyz[g, 0:16]
            for k in range(K):
                r = g * K + k
                outbuf[r, 0:16] = xrows[r, 0:16] - cpat
                for j in range(C // 16):
                    src = frows[r, 16 * j:16 * (j + 1)]
                    cf = cfeat[g, 16 * j:16 * (j + 1)]
                    outbuf[r, 3 + 16 * j:3 + 16 * (j + 1)] = src
                    outbuf[r, 67 + 16 * j:67 + 16 * (j + 1)] = src - cf

        pltpu.sync_copy(outbuf, out_hbm.at[pl.ds(g0 * K, _ROWS)])
        return 0

    lax.fori_loop(0, _NCHUNK, chunk, 0)


@functools.partial(
    pl.kernel,
    out_type=jax.ShapeDtypeStruct((B * G * K, OUTC), jnp.float32),
    mesh=plsc.VectorSubcoreMesh(core_axis_name="c", subcore_axis_name="s",
                                num_cores=_NC),
    compiler_params=pltpu.CompilerParams(use_tc_tiling_on_sc=False),
    scratch_types=[
        pltpu.VMEM((_ROWS,), jnp.int32),
        pltpu.VMEM((_ROWS, C), jnp.float32),
        pltpu.VMEM((_ROWS, 16), jnp.float32),
        pltpu.VMEM((_GC,), jnp.int32),
        pltpu.VMEM((_GC, C), jnp.float32),
        pltpu.VMEM((_GC, 16), jnp.float32),
        pltpu.VMEM((_ROWS, OUTC), jnp.float32),
        pltpu.SemaphoreType.DMA,
    ],
)
def _group_sc(feats_hbm, xyzp_hbm, knn_hbm, fps_hbm, cent_hbm, out_hbm,
              idx_v, frows, xrows, cfidx, cfeat, cxyz, outbuf, sem):
    _group_body(feats_hbm, xyzp_hbm, knn_hbm, fps_hbm, cent_hbm, out_hbm,
                idx_v, frows, xrows, cfidx, cfeat, cxyz, outbuf, sem)


def kernel(xyz, features):
    batch_size, num_points, _ = xyz.shape
    xyz_t = jnp.transpose(xyz, (2, 0, 1))
    fps_idx, centers_t = _fps_pallas(xyz_t)
    centers = jnp.transpose(centers_t, (1, 2, 0))

    pad1 = jnp.zeros(xyz.shape[:-1] + (1,), xyz.dtype)
    keys4 = jnp.concatenate([xyz, pad1], axis=-1)
    q4t = jnp.concatenate(
        [centers_t, jnp.zeros((1, B, G), xyz.dtype)], axis=0)
    q4t = jnp.transpose(q4t, (1, 0, 2))
    knnT = _knn_pallas(keys4, keys4.astype(jnp.bfloat16), q4t,
                       q4t.astype(jnp.bfloat16))
    knn_idx = jnp.transpose(knnT, (0, 2, 1))

    batch_offset = (jnp.arange(batch_size, dtype=jnp.int32) * num_points)
    knn_flat = (knn_idx + batch_offset[:, None, None]).reshape(-1)
    fps_flat = (fps_idx + batch_offset[:, None]).reshape(-1)

    feats_flat = features.reshape(B * N, C)
    xyzp = jnp.concatenate(
        [xyz.reshape(B * N, 3), jnp.zeros((B * N, 13), xyz.dtype)], -1)
    cent_pad = jnp.concatenate(
        [centers.reshape(B * G, 3), jnp.zeros((B * G, 13), xyz.dtype)], -1)

    out = _group_sc(feats_flat, xyzp, knn_flat, fps_flat, cent_pad)
    group_feats = out.reshape(B, G, K, OUTC)
    return group_feats, centers, knn_idx, fps_idx


# speedup vs baseline: 5.0426x; 1.0383x over previous
import functools

import jax
import jax.numpy as jnp
from jax import lax
from jax.experimental import pallas as pl
from jax.experimental.pallas import tpu as pltpu
from jax.experimental.pallas import tpu_sc as plsc

B, N, C = 8, 8192, 64
G = 512
K = 32
OUTC = 3 + C + C


def _fps_body(x_ref, fps_ref, c_ref, dists_ref):
    px = x_ref[0]
    py = x_ref[1]
    pz = x_ref[2]
    lane = lax.broadcasted_iota(jnp.int32, (B, N), 1)
    col = lax.broadcasted_iota(jnp.int32, (B, G), 1)

    fps_ref[...] = jnp.zeros((B, G), jnp.int32)
    dists_ref[...] = jnp.full((B, N), jnp.inf, jnp.float32)
    lx0, ly0, lz0 = px[:, 0:1], py[:, 0:1], pz[:, 0:1]
    c_ref[0] = jnp.where(col == 0, lx0, 0.0)
    c_ref[1] = jnp.where(col == 0, ly0, 0.0)
    c_ref[2] = jnp.where(col == 0, lz0, 0.0)

    def body(i, carry):
        lx, ly, lz = carry
        dx = px - lx
        dy = py - ly
        dz = pz - lz
        d = (dx * dx + dy * dy) + dz * dz
        dists = jnp.minimum(dists_ref[...], d)
        dists_ref[...] = dists
        m = jnp.max(dists, axis=1, keepdims=True)
        cand = jnp.where(dists == m, lane, N)
        nxt = jnp.min(cand, axis=1, keepdims=True)
        fps_ref[...] = jnp.where(col == i, nxt, fps_ref[...])
        eq = lane == nxt
        lx2 = jnp.sum(jnp.where(eq, px, 0.0), axis=1, keepdims=True)
        ly2 = jnp.sum(jnp.where(eq, py, 0.0), axis=1, keepdims=True)
        lz2 = jnp.sum(jnp.where(eq, pz, 0.0), axis=1, keepdims=True)
        c_ref[0] = jnp.where(col == i, lx2, c_ref[0])
        c_ref[1] = jnp.where(col == i, ly2, c_ref[1])
        c_ref[2] = jnp.where(col == i, lz2, c_ref[2])
        return (lx2, ly2, lz2)

    lax.fori_loop(1, G, body, (lx0, ly0, lz0))


def _fps_pallas(xyz_t):
    return pl.pallas_call(
        _fps_body,
        out_shape=[
            jax.ShapeDtypeStruct((B, G), jnp.int32),
            jax.ShapeDtypeStruct((3, B, G), jnp.float32),
        ],
        scratch_shapes=[pltpu.VMEM((B, N), jnp.float32)],
    )(xyz_t)


_NCH = 4
_CH = N // _NCH


def _knn_body(k_ref, kb_ref, q_ref, qb_ref, out_ref, d_ref):
    qx = q_ref[0, 0:1, :]
    qy = q_ref[0, 1:2, :]
    qz = q_ref[0, 2:3, :]
    qq = (qx * qx + qy * qy) + qz * qz

    m0 = jnp.full((1, G), jnp.inf, jnp.float32)
    for c in range(_NCH):
        kc = k_ref[0, pl.ds(c * _CH, _CH), :]
        kx = kc[:, 0:1]
        ky = kc[:, 1:2]
        kz = kc[:, 2:3]
        kk = (kx * kx + ky * ky) + kz * kz
        dot = lax.dot_general(kb_ref[0, pl.ds(c * _CH, _CH), :], qb_ref[0],
                              (((1,), (0,)), ((), ())),
                              preferred_element_type=jnp.float32)
        d2 = (qq + kk) - 2.0 * dot
        dc = jnp.sqrt(jnp.maximum(d2, 0.0))
        d_ref[pl.ds(c * _CH, _CH), :] = dc
        m0 = jnp.minimum(m0, jnp.min(dc, axis=0, keepdims=True))

    rows0 = lax.broadcasted_iota(jnp.int32, (_CH, 1), 0)

    def body(j, m):
        idx = jnp.full((1, G), N, jnp.int32)
        for c in range(_NCH):
            dv = d_ref[pl.ds(c * _CH, _CH), :]
            cand = jnp.where(dv == m, rows0 + (c * _CH), N)
            idx = jnp.minimum(idx, jnp.min(cand, axis=0, keepdims=True))
        out_ref[0, pl.ds(j, 1), :] = idx
        m = jnp.full((1, G), jnp.inf, jnp.float32)
        for c in range(_NCH):
            dv = d_ref[pl.ds(c * _CH, _CH), :]
            hit = (rows0 + (c * _CH)) == idx
            dv = jnp.where(hit, jnp.inf, dv)
            d_ref[pl.ds(c * _CH, _CH), :] = dv
            m = jnp.minimum(m, jnp.min(dv, axis=0, keepdims=True))
        return m

    lax.fori_loop(0, K, body, m0)


def _knn_pallas(keys4, keys4_bf, q4t, q4t_bf):
    return pl.pallas_call(
        _knn_body,
        grid=(B,),
        in_specs=[
            pl.BlockSpec((1, N, 4), lambda b: (b, 0, 0)),
            pl.BlockSpec((1, N, 4), lambda b: (b, 0, 0)),
            pl.BlockSpec((1, 4, G), lambda b: (b, 0, 0)),
            pl.BlockSpec((1, 4, G), lambda b: (b, 0, 0)),
        ],
        out_specs=pl.BlockSpec((1, K, G), lambda b: (b, 0, 0)),
        out_shape=jax.ShapeDtypeStruct((B, K, G), jnp.int32),
        scratch_shapes=[pltpu.VMEM((N, G), jnp.float32)],
    )(keys4, keys4_bf, q4t, q4t_bf)


_NC, _NS = 2, 16
_NW = _NC * _NS
_GPW = (B * G) // _NW
_GC = 8
_NCHUNK = _GPW // _GC
_ROWS = _GC * K


def _group_body(feats_hbm, xyzp_hbm, knn_hbm, fps_hbm, cent_hbm, out_hbm,
                idx_v, frows, xrows, cfidx, cfeat, cxyz, outbuf, sem):
    wid = lax.axis_index("s") * _NC + lax.axis_index("c")

    def chunk(ch, _):
        g0 = wid * _GPW + ch * _GC
        pltpu.sync_copy(knn_hbm.at[pl.ds(g0 * K, _ROWS)], idx_v)
        pltpu.sync_copy(fps_hbm.at[pl.ds(g0, _GC)], cfidx)
        pltpu.sync_copy(cent_hbm.at[pl.ds(g0, _GC)], cxyz)
        pltpu.async_copy(feats_hbm.at[idx_v], frows, sem).wait()
        pltpu.async_copy(xyzp_hbm.at[idx_v], xrows, sem).wait()
        pltpu.async_copy(feats_hbm.at[cfidx], cfeat, sem).wait()

        for g in range(_GC):
            cpat = cxyz[g, 0:16]
            for k in range(K):
                r = g * K + k
                outbuf[r, 0:16] = xrows[r, 0:16] - cpat
                for j in range(C // 16):
                    src = frows[r, 16 * j:16 * (j + 1)]
                    cf = cfeat[g, 16 * j:16 * (j + 1)]
                    outbuf[r, 3 + 16 * j:3 + 16 * (j + 1)] = src
                    outbuf[r, 67 + 16 * j:67 + 16 * (j + 1)] = src - cf

        pltpu.sync_copy(outbuf, out_hbm.at[pl.ds(g0 * K, _ROWS)])
        return 0

    lax.fori_loop(0, _NCHUNK, chunk, 0)


@functools.partial(
    pl.kernel,
    out_type=jax.ShapeDtypeStruct((B * G * K, OUTC), jnp.float32),
    mesh=plsc.VectorSubcoreMesh(core_axis_name="c", subcore_axis_name="s",
                                num_cores=_NC),
    compiler_params=pltpu.CompilerParams(use_tc_tiling_on_sc=False),
    scratch_types=[
        pltpu.VMEM((_ROWS,), jnp.int32),
        pltpu.VMEM((_ROWS, C), jnp.float32),
        pltpu.VMEM((_ROWS, 16), jnp.float32),
        pltpu.VMEM((_GC,), jnp.int32),
        pltpu.VMEM((_GC, C), jnp.float32),
        pltpu.VMEM((_GC, 16), jnp.float32),
        pltpu.VMEM((_ROWS, OUTC), jnp.float32),
        pltpu.SemaphoreType.DMA,
    ],
)
def _group_sc(feats_hbm, xyzp_hbm, knn_hbm, fps_hbm, cent_hbm, out_hbm,
              idx_v, frows, xrows, cfidx, cfeat, cxyz, outbuf, sem):
    _group_body(feats_hbm, xyzp_hbm, knn_hbm, fps_hbm, cent_hbm, out_hbm,
                idx_v, frows, xrows, cfidx, cfeat, cxyz, outbuf, sem)


def kernel(xyz, features):
    batch_size, num_points, _ = xyz.shape
    xyz_t = jnp.transpose(xyz, (2, 0, 1))
    fps_idx, centers_t = _fps_pallas(xyz_t)
    centers = jnp.transpose(centers_t, (1, 2, 0))

    pad1 = jnp.zeros(xyz.shape[:-1] + (1,), xyz.dtype)
    keys4 = jnp.concatenate([xyz, pad1], axis=-1)
    q4t = jnp.concatenate(
        [centers_t, jnp.zeros((1, B, G), xyz.dtype)], axis=0)
    q4t = jnp.transpose(q4t, (1, 0, 2))
    knnT = _knn_pallas(keys4, keys4.astype(jnp.bfloat16), q4t,
                       q4t.astype(jnp.bfloat16))
    knn_idx = jnp.transpose(knnT, (0, 2, 1))

    batch_offset = (jnp.arange(batch_size, dtype=jnp.int32) * num_points)
    knn_flat = (knn_idx + batch_offset[:, None, None]).reshape(-1)
    fps_flat = (fps_idx + batch_offset[:, None]).reshape(-1)

    feats_flat = features.reshape(B * N, C)
    xyzp = jnp.concatenate(
        [xyz.reshape(B * N, 3), jnp.zeros((B * N, 13), xyz.dtype)], -1)
    cent_pad = jnp.concatenate(
        [centers.reshape(B * G, 3), jnp.zeros((B * G, 13), xyz.dtype)], -1)

    out = _group_sc(feats_flat, xyzp, knn_flat, fps_flat, cent_pad)
    group_feats = out.reshape(B, G, K, OUTC)
    return group_feats, centers, knn_idx, fps_idx


# speedup vs baseline: 5.1180x; 1.0149x over previous
import functools

import jax
import jax.numpy as jnp
from jax import lax
from jax.experimental import pallas as pl
from jax.experimental.pallas import tpu as pltpu
from jax.experimental.pallas import tpu_sc as plsc

B, N, C = 8, 8192, 64
G = 512
K = 32
OUTC = 3 + C + C


def _fps_body(x_ref, fps_ref, c_ref, dists_ref):
    px = x_ref[0]
    py = x_ref[1]
    pz = x_ref[2]
    lane = lax.broadcasted_iota(jnp.int32, (B, N), 1)
    col = lax.broadcasted_iota(jnp.int32, (B, G), 1)

    fps_ref[...] = jnp.zeros((B, G), jnp.int32)
    dists_ref[...] = jnp.full((B, N), jnp.inf, jnp.float32)
    lx0, ly0, lz0 = px[:, 0:1], py[:, 0:1], pz[:, 0:1]
    c_ref[0] = jnp.where(col == 0, lx0, 0.0)
    c_ref[1] = jnp.where(col == 0, ly0, 0.0)
    c_ref[2] = jnp.where(col == 0, lz0, 0.0)

    def body(i, carry):
        lx, ly, lz = carry
        dx = px - lx
        dy = py - ly
        dz = pz - lz
        d = (dx * dx + dy * dy) + dz * dz
        dists = jnp.minimum(dists_ref[...], d)
        dists_ref[...] = dists
        m = jnp.max(dists, axis=1, keepdims=True)
        cand = jnp.where(dists == m, lane, N)
        nxt = jnp.min(cand, axis=1, keepdims=True)
        fps_ref[...] = jnp.where(col == i, nxt, fps_ref[...])
        eq = lane == nxt
        lx2 = jnp.sum(jnp.where(eq, px, 0.0), axis=1, keepdims=True)
        ly2 = jnp.sum(jnp.where(eq, py, 0.0), axis=1, keepdims=True)
        lz2 = jnp.sum(jnp.where(eq, pz, 0.0), axis=1, keepdims=True)
        c_ref[0] = jnp.where(col == i, lx2, c_ref[0])
        c_ref[1] = jnp.where(col == i, ly2, c_ref[1])
        c_ref[2] = jnp.where(col == i, lz2, c_ref[2])
        return (lx2, ly2, lz2)

    lax.fori_loop(1, G, body, (lx0, ly0, lz0))


def _fps_pallas(xyz_t):
    return pl.pallas_call(
        _fps_body,
        out_shape=[
            jax.ShapeDtypeStruct((B, G), jnp.int32),
            jax.ShapeDtypeStruct((3, B, G), jnp.float32),
        ],
        scratch_shapes=[pltpu.VMEM((B, N), jnp.float32)],
    )(xyz_t)


_NCH = 4
_CH = N // _NCH


def _knn_body(k_ref, kb_ref, q_ref, qb_ref, out_ref, d_ref):
    qx = q_ref[0, 0:1, :]
    qy = q_ref[0, 1:2, :]
    qz = q_ref[0, 2:3, :]
    qq = (qx * qx + qy * qy) + qz * qz

    m0 = jnp.full((1, G), jnp.inf, jnp.float32)
    for c in range(_NCH):
        kc = k_ref[0, pl.ds(c * _CH, _CH), :]
        kx = kc[:, 0:1]
        ky = kc[:, 1:2]
        kz = kc[:, 2:3]
        kk = (kx * kx + ky * ky) + kz * kz
        dot = lax.dot_general(kb_ref[0, pl.ds(c * _CH, _CH), :], qb_ref[0],
                              (((1,), (0,)), ((), ())),
                              preferred_element_type=jnp.float32)
        d2 = (qq + kk) - 2.0 * dot
        dc = jnp.sqrt(jnp.maximum(d2, 0.0))
        d_ref[pl.ds(c * _CH, _CH), :] = dc
        m0 = jnp.minimum(m0, jnp.min(dc, axis=0, keepdims=True))

    rows0 = lax.broadcasted_iota(jnp.int32, (_CH, 1), 0)

    def body(j, m):
        idx = jnp.full((1, G), N, jnp.int32)
        for c in range(_NCH):
            dv = d_ref[pl.ds(c * _CH, _CH), :]
            cand = jnp.where(dv == m, rows0 + (c * _CH), N)
            idx = jnp.minimum(idx, jnp.min(cand, axis=0, keepdims=True))
        out_ref[0, pl.ds(j, 1), :] = idx
        m = jnp.full((1, G), jnp.inf, jnp.float32)
        for c in range(_NCH):
            dv = d_ref[pl.ds(c * _CH, _CH), :]
            hit = (rows0 + (c * _CH)) == idx
            dv = jnp.where(hit, jnp.inf, dv)
            d_ref[pl.ds(c * _CH, _CH), :] = dv
            m = jnp.minimum(m, jnp.min(dv, axis=0, keepdims=True))
        return m

    lax.fori_loop(0, K, body, m0)


def _knn_pallas(keys4, keys4_bf, q4t, q4t_bf):
    return pl.pallas_call(
        _knn_body,
        grid=(B,),
        in_specs=[
            pl.BlockSpec((1, N, 4), lambda b: (b, 0, 0)),
            pl.BlockSpec((1, N, 4), lambda b: (b, 0, 0)),
            pl.BlockSpec((1, 4, G), lambda b: (b, 0, 0)),
            pl.BlockSpec((1, 4, G), lambda b: (b, 0, 0)),
        ],
        out_specs=pl.BlockSpec((1, K, G), lambda b: (b, 0, 0)),
        out_shape=jax.ShapeDtypeStruct((B, K, G), jnp.int32),
        scratch_shapes=[pltpu.VMEM((N, G), jnp.float32)],
    )(keys4, keys4_bf, q4t, q4t_bf)


_NC, _NS = 2, 16
_NW = _NC * _NS
_GPW = (B * G) // _NW
_GC = 8
_NCHUNK = _GPW // _GC
_ROWS = _GC * K


def _group_body(feats_hbm, xyzp_hbm, knn_hbm, fps_hbm, cent_hbm, out_hbm,
                idx_v, frows, xrows, cfidx, cfeat, cxyz, outbuf, sem, osem):
    wid = lax.axis_index("s") * _NC + lax.axis_index("c")

    def chunk(ch, _):
        g0 = wid * _GPW + ch * _GC
        pltpu.sync_copy(knn_hbm.at[pl.ds(g0 * K, _ROWS)], idx_v)
        pltpu.sync_copy(fps_hbm.at[pl.ds(g0, _GC)], cfidx)
        pltpu.sync_copy(cent_hbm.at[pl.ds(g0, _GC)], cxyz)
        c1 = pltpu.async_copy(feats_hbm.at[idx_v], frows, sem)
        c2 = pltpu.async_copy(xyzp_hbm.at[idx_v], xrows, sem)
        c3 = pltpu.async_copy(feats_hbm.at[cfidx], cfeat, sem)

        @pl.when(ch > 0)
        def _():
            pltpu.make_async_copy(
                out_hbm.at[pl.ds(0, _ROWS)], outbuf, osem).wait()

        c1.wait()
        c2.wait()
        c3.wait()

        for g in range(_GC):
            cpat = cxyz[g, 0:16]
            for k in range(K):
                r = g * K + k
                outbuf[r, 0:16] = xrows[r, 0:16] - cpat
                for j in range(C // 16):
                    src = frows[r, 16 * j:16 * (j + 1)]
                    cf = cfeat[g, 16 * j:16 * (j + 1)]
                    outbuf[r, 3 + 16 * j:3 + 16 * (j + 1)] = src
                    outbuf[r, 67 + 16 * j:67 + 16 * (j + 1)] = src - cf

        pltpu.async_copy(outbuf, out_hbm.at[pl.ds(g0 * K, _ROWS)], osem)
        return 0

    lax.fori_loop(0, _NCHUNK, chunk, 0)
    pltpu.make_async_copy(out_hbm.at[pl.ds(0, _ROWS)], outbuf, osem).wait()


@functools.partial(
    pl.kernel,
    out_type=jax.ShapeDtypeStruct((B * G * K, OUTC), jnp.float32),
    mesh=plsc.VectorSubcoreMesh(core_axis_name="c", subcore_axis_name="s",
                                num_cores=_NC),
    compiler_params=pltpu.CompilerParams(use_tc_tiling_on_sc=False),
    scratch_types=[
        pltpu.VMEM((_ROWS,), jnp.int32),
        pltpu.VMEM((_ROWS, C), jnp.float32),
        pltpu.VMEM((_ROWS, 16), jnp.float32),
        pltpu.VMEM((_GC,), jnp.int32),
        pltpu.VMEM((_GC, C), jnp.float32),
        pltpu.VMEM((_GC, 16), jnp.float32),
        pltpu.VMEM((_ROWS, OUTC), jnp.float32),
        pltpu.SemaphoreType.DMA,
        pltpu.SemaphoreType.DMA,
    ],
)
def _group_sc(feats_hbm, xyzp_hbm, knn_hbm, fps_hbm, cent_hbm, out_hbm,
              idx_v, frows, xrows, cfidx, cfeat, cxyz, outbuf, sem, osem):
    _group_body(feats_hbm, xyzp_hbm, knn_hbm, fps_hbm, cent_hbm, out_hbm,
                idx_v, frows, xrows, cfidx, cfeat, cxyz, outbuf, sem, osem)


def kernel(xyz, features):
    batch_size, num_points, _ = xyz.shape
    xyz_t = jnp.transpose(xyz, (2, 0, 1))
    fps_idx, centers_t = _fps_pallas(xyz_t)
    centers = jnp.transpose(centers_t, (1, 2, 0))

    pad1 = jnp.zeros(xyz.shape[:-1] + (1,), xyz.dtype)
    keys4 = jnp.concatenate([xyz, pad1], axis=-1)
    q4t = jnp.concatenate(
        [centers_t, jnp.zeros((1, B, G), xyz.dtype)], axis=0)
    q4t = jnp.transpose(q4t, (1, 0, 2))
    knnT = _knn_pallas(keys4, keys4.astype(jnp.bfloat16), q4t,
                       q4t.astype(jnp.bfloat16))
    knn_idx = jnp.transpose(knnT, (0, 2, 1))

    batch_offset = (jnp.arange(batch_size, dtype=jnp.int32) * num_points)
    knn_flat = (knn_idx + batch_offset[:, None, None]).reshape(-1)
    fps_flat = (fps_idx + batch_offset[:, None]).reshape(-1)

    feats_flat = features.reshape(B * N, C)
    xyzp = jnp.concatenate(
        [xyz.reshape(B * N, 3), jnp.zeros((B * N, 13), xyz.dtype)], -1)
    cent_pad = jnp.concatenate(
        [centers.reshape(B * G, 3), jnp.zeros((B * G, 13), xyz.dtype)], -1)

    out = _group_sc(feats_flat, xyzp, knn_flat, fps_flat, cent_pad)
    group_feats = out.reshape(B, G, K, OUTC)
    return group_feats, centers, knn_idx, fps_idx
